# Initial kernel scaffold; baseline (speedup 1.0000x reference)
import functools

import jax
import jax.numpy as jnp
from jax import lax
from jax.experimental import pallas as pl
from jax.experimental.pallas import tpu as pltpu
from jax.experimental.pallas import tpu_sc as plsc

NC = 2
NS = 16
K = 80
CW = 16


def _proj_body(x_ref, wT_ref, b_ref, o_ref):
    h = jnp.dot(x_ref[...], wT_ref[...], preferred_element_type=jnp.float32)
    o_ref[...] = jnp.maximum(h + b_ref[...], 0.0)


@functools.lru_cache(maxsize=None)
def _make_proj(N, D, RB):
    return pl.pallas_call(
        _proj_body,
        grid=(N // RB,),
        in_specs=[
            pl.BlockSpec((RB, D), lambda i: (i, 0)),
            pl.BlockSpec((D, D), lambda i: (0, 0)),
            pl.BlockSpec((1, D), lambda i: (0, 0)),
        ],
        out_specs=pl.BlockSpec((RB, D), lambda i: (i, 0)),
        out_shape=jax.ShapeDtypeStruct((N, D), jnp.float32),
    )


@functools.lru_cache(maxsize=None)
def _make_sc_aggregate(N, D, E):
    rpt = N // NS
    e_per_tile = E // (NC * NS)
    n_chunks = e_per_tile // K
    assert rpt * NS == N and n_chunks * K == e_per_tile

    mesh = plsc.VectorSubcoreMesh(core_axis_name="c", subcore_axis_name="s")

    def body(h_hbm, src_hbm, dst_hbm, zf_hbm, zc_hbm, ones_hbm,
             agg_out, cnt_out,
             src_v, dst_v, rows_v, ones_v, acc_sp, cnt_sp, sem):
        c = lax.axis_index("c")
        s = lax.axis_index("s")
        r0 = s * rpt
        pltpu.sync_copy(zf_hbm, acc_sp.at[pl.ds(r0, rpt)])
        pltpu.sync_copy(zc_hbm, cnt_sp.at[pl.ds(r0, rpt)])
        pltpu.sync_copy(ones_hbm, ones_v)
        plsc.subcore_barrier()

        base = c * (E // NC) + s * e_per_tile

        def chunk(i, carry):
            off = pl.multiple_of(base + i * K, K)
            pltpu.sync_copy(src_hbm.at[pl.ds(off, K)], src_v)
            pltpu.sync_copy(dst_hbm.at[pl.ds(off, K)], dst_v)
            pltpu.async_copy(h_hbm.at[src_v], rows_v, sem).wait()
            pltpu.sync_copy(rows_v, acc_sp.at[dst_v], add=True)
            pltpu.sync_copy(ones_v, cnt_sp.at[dst_v], add=True)
            return carry

        lax.fori_loop(0, n_chunks, chunk, 0)
        plsc.subcore_barrier()

        q0 = c * N + r0
        pltpu.sync_copy(acc_sp.at[pl.ds(r0, rpt)], agg_out.at[pl.ds(q0, rpt)])
        pltpu.sync_copy(cnt_sp.at[pl.ds(r0, rpt)], cnt_out.at[pl.ds(q0, rpt)])

    return pl.kernel(
        body,
        out_type=[
            jax.ShapeDtypeStruct((NC * N, D), jnp.float32),
            jax.ShapeDtypeStruct((NC * N, CW), jnp.float32),
        ],
        mesh=mesh,
        scratch_types=[
            pltpu.VMEM((K,), jnp.int32),
            pltpu.VMEM((K,), jnp.int32),
            pltpu.VMEM((K, D), jnp.float32),
            pltpu.VMEM((K, CW), jnp.float32),
            pltpu.VMEM_SHARED((N, D), jnp.float32),
            pltpu.VMEM_SHARED((N, CW), jnp.float32),
            pltpu.SemaphoreType.DMA,
        ],
    )


def _post_body(aggp_ref, cntp_ref, x_ref, wlT_ref, bl_ref, wrT_ref, br_ref,
               o_ref):
    agg = aggp_ref[0] + aggp_ref[1]
    cnt = cntp_ref[0, :, 0:1] + cntp_ref[1, :, 0:1]
    mean = agg / jnp.maximum(cnt, 1.0)
    x = x_ref[...]
    out = (jnp.dot(mean, wlT_ref[...], preferred_element_type=jnp.float32)
           + bl_ref[...]
           + jnp.dot(x, wrT_ref[...], preferred_element_type=jnp.float32)
           + br_ref[...])
    out = jnp.where(out > 0, out, jnp.expm1(out))
    norm = jnp.maximum(jnp.sqrt(jnp.sum(out * out, axis=1, keepdims=True)),
                       1e-12)
    o_ref[...] = x + out / norm


@functools.lru_cache(maxsize=None)
def _make_post(N, D, RB):
    return pl.pallas_call(
        _post_body,
        grid=(N // RB,),
        in_specs=[
            pl.BlockSpec((NC, RB, D), lambda i: (0, i, 0)),
            pl.BlockSpec((NC, RB, CW), lambda i: (0, i, 0)),
            pl.BlockSpec((RB, D), lambda i: (i, 0)),
            pl.BlockSpec((D, D), lambda i: (0, 0)),
            pl.BlockSpec((1, D), lambda i: (0, 0)),
            pl.BlockSpec((D, D), lambda i: (0, 0)),
            pl.BlockSpec((1, D), lambda i: (0, 0)),
        ],
        out_specs=pl.BlockSpec((RB, D), lambda i: (i, 0)),
        out_shape=jax.ShapeDtypeStruct((N, D), jnp.float32),
    )


def kernel(x, edge_index, W_proj, b_proj, W_l, b_l, W_r, b_r):
    N, D = x.shape
    E = edge_index.shape[1]
    src = edge_index[0].astype(jnp.int32)
    dst = edge_index[1].astype(jnp.int32)

    h = _make_proj(N, D, 1000)(x, W_proj.T, b_proj.reshape(1, D))

    zf = jnp.zeros((N // NS, D), jnp.float32)
    zc = jnp.zeros((N // NS, CW), jnp.float32)
    ones = jnp.ones((K, CW), jnp.float32)
    aggp, cntp = _make_sc_aggregate(N, D, E)(h, src, dst, zf, zc, ones)

    out = _make_post(N, D, 1000)(
        aggp.reshape(NC, N, D), cntp.reshape(NC, N, CW),
        x, W_l.T, b_l.reshape(1, D), W_r.T, b_r.reshape(1, D))
    return out


# speedup vs baseline: 5.8946x; 5.8946x over previous
import functools

import jax
import jax.numpy as jnp
from jax import lax
from jax.experimental import pallas as pl
from jax.experimental.pallas import tpu as pltpu
from jax.experimental.pallas import tpu_sc as plsc

NC = 2
NS = 16
K = 80
CW = 16


def _proj_body(x_ref, wT_ref, b_ref, o_ref):
    h = jnp.dot(x_ref[...], wT_ref[...], preferred_element_type=jnp.float32)
    o_ref[...] = jnp.maximum(h + b_ref[...], 0.0)


@functools.lru_cache(maxsize=None)
def _make_proj(N, D, RB):
    return pl.pallas_call(
        _proj_body,
        grid=(N // RB,),
        in_specs=[
            pl.BlockSpec((RB, D), lambda i: (i, 0)),
            pl.BlockSpec((D, D), lambda i: (0, 0)),
            pl.BlockSpec((1, D), lambda i: (0, 0)),
        ],
        out_specs=pl.BlockSpec((RB, D), lambda i: (i, 0)),
        out_shape=jax.ShapeDtypeStruct((N, D), jnp.float32),
    )


@functools.lru_cache(maxsize=None)
def _make_sc_aggregate(N, D, E):
    NP = -(-N // (NS * 8)) * (NS * 8)
    rpt = NP // NS
    e_per_tile = E // (NC * NS)
    n_chunks = e_per_tile // K
    assert n_chunks * K == e_per_tile

    mesh = plsc.VectorSubcoreMesh(core_axis_name="c", subcore_axis_name="s")

    def body(h_hbm, src_hbm, dst_hbm, zf_hbm,
             agg_out, cnt_out,
             src_v, dst_v, rows_v, ones_v, zc_v, acc_sp, cnt_sp, sem):
        c = lax.axis_index("c")
        s = lax.axis_index("s")
        r0 = s * rpt
        pltpu.sync_copy(zf_hbm, acc_sp.at[pl.ds(r0, rpt)])

        def zfill(i, carry):
            zc_v[pl.ds(i * 16, 16)] = jnp.zeros((16,), jnp.float32)
            return carry

        lax.fori_loop(0, rpt // 16, zfill, 0)
        for j in range(K // 16):
            ones_v[pl.ds(j * 16, 16)] = jnp.ones((16,), jnp.float32)
        pltpu.sync_copy(zc_v, cnt_sp.at[pl.ds(r0, rpt)])
        plsc.subcore_barrier()

        base = c * (E // NC) + s * e_per_tile

        def chunk(i, carry):
            off = pl.multiple_of(base + i * K, K)
            pltpu.sync_copy(src_hbm.at[pl.ds(off, K)], src_v)
            pltpu.sync_copy(dst_hbm.at[pl.ds(off, K)], dst_v)
            pltpu.async_copy(h_hbm.at[src_v], rows_v, sem).wait()
            pltpu.sync_copy(rows_v, acc_sp.at[dst_v], add=True)
            pltpu.sync_copy(ones_v, cnt_sp.at[dst_v], add=True)
            return carry

        lax.fori_loop(0, n_chunks, chunk, 0)
        plsc.subcore_barrier()

        q0 = c * NP + r0
        pltpu.sync_copy(acc_sp.at[pl.ds(r0, rpt)], agg_out.at[pl.ds(q0, rpt)])
        pltpu.sync_copy(cnt_sp.at[pl.ds(r0, rpt)], zc_v)
        w = c * NS + s
        pltpu.sync_copy(zc_v, cnt_out.at[w])

    return pl.kernel(
        body,
        out_type=[
            jax.ShapeDtypeStruct((NC * NP, D), jnp.float32),
            jax.ShapeDtypeStruct((NC * NS, rpt), jnp.float32),
        ],
        mesh=mesh,
        scratch_types=[
            pltpu.VMEM((K,), jnp.int32),
            pltpu.VMEM((K,), jnp.int32),
            pltpu.VMEM((K, D), jnp.float32),
            pltpu.VMEM((K,), jnp.float32),
            pltpu.VMEM((rpt,), jnp.float32),
            pltpu.VMEM_SHARED((NP, D), jnp.float32),
            pltpu.VMEM_SHARED((NP,), jnp.float32),
            pltpu.SemaphoreType.DMA,
        ],
    ), NP


def _post_body(aggp_ref, cntp_ref, x_ref, wlT_ref, bl_ref, wrT_ref, br_ref,
               o_ref):
    agg = aggp_ref[0] + aggp_ref[1]
    cnt = jnp.sum(cntp_ref[...], axis=0)
    mean = agg / jnp.maximum(cnt, 1.0)
    x = x_ref[...]
    out = (jnp.dot(mean, wlT_ref[...], preferred_element_type=jnp.float32)
           + bl_ref[...]
           + jnp.dot(x, wrT_ref[...], preferred_element_type=jnp.float32)
           + br_ref[...])
    out = jnp.where(out > 0, out, jnp.exp(jnp.minimum(out, 0.0)) - 1.0)
    norm = jnp.maximum(jnp.sqrt(jnp.sum(out * out, axis=1, keepdims=True)),
                       1e-12)
    o_ref[...] = x + out / norm


@functools.lru_cache(maxsize=None)
def _make_post(N, D, RB):
    return pl.pallas_call(
        _post_body,
        grid=(N // RB,),
        in_specs=[
            pl.BlockSpec((NC, RB, D), lambda i: (0, i, 0)),
            pl.BlockSpec((NC, RB, 1), lambda i: (0, i, 0)),
            pl.BlockSpec((RB, D), lambda i: (i, 0)),
            pl.BlockSpec((D, D), lambda i: (0, 0)),
            pl.BlockSpec((1, D), lambda i: (0, 0)),
            pl.BlockSpec((D, D), lambda i: (0, 0)),
            pl.BlockSpec((1, D), lambda i: (0, 0)),
        ],
        out_specs=pl.BlockSpec((RB, D), lambda i: (i, 0)),
        out_shape=jax.ShapeDtypeStruct((N, D), jnp.float32),
    )


def kernel(x, edge_index, W_proj, b_proj, W_l, b_l, W_r, b_r):
    N, D = x.shape
    E = edge_index.shape[1]
    src = edge_index[0].astype(jnp.int32)
    dst = edge_index[1].astype(jnp.int32)

    h = _make_proj(N, D, 1000)(x, W_proj.T, b_proj.reshape(1, D))

    sc_call, NP = _make_sc_aggregate(N, D, E)
    zf = jnp.zeros((NP // NS, D), jnp.float32)
    aggp, cntp = sc_call(h, src, dst, zf)

    out = _make_post(N, D, 1000)(
        aggp.reshape(NC, NP, D), cntp.reshape(NC, NP, 1),
        x, W_l.T, b_l.reshape(1, D), W_r.T, b_r.reshape(1, D))
    return out


# speedup vs baseline: 13.8905x; 2.3565x over previous
import functools

import jax
import jax.numpy as jnp
from jax import lax
from jax.experimental import pallas as pl
from jax.experimental.pallas import tpu as pltpu
from jax.experimental.pallas import tpu_sc as plsc

NC = 2
NS = 16
K = 80
CW = 16


def _proj_body(x_ref, wT_ref, b_ref, o_ref):
    h = jnp.dot(x_ref[...], wT_ref[...], preferred_element_type=jnp.float32)
    o_ref[...] = jnp.maximum(h + b_ref[...], 0.0)


@functools.lru_cache(maxsize=None)
def _make_proj(N, D, RB):
    return pl.pallas_call(
        _proj_body,
        grid=(N // RB,),
        in_specs=[
            pl.BlockSpec((RB, D), lambda i: (i, 0)),
            pl.BlockSpec((D, D), lambda i: (0, 0)),
            pl.BlockSpec((1, D), lambda i: (0, 0)),
        ],
        out_specs=pl.BlockSpec((RB, D), lambda i: (i, 0)),
        out_shape=jax.ShapeDtypeStruct((N, D), jnp.float32),
    )


@functools.lru_cache(maxsize=None)
def _make_sc_aggregate(N, D, E):
    NP = -(-N // (NS * 8)) * (NS * 8)
    rpt = NP // NS
    e_per_tile = E // (NC * NS)
    n_chunks = e_per_tile // K
    assert n_chunks * K == e_per_tile

    mesh = plsc.VectorSubcoreMesh(core_axis_name="c", subcore_axis_name="s")

    NB = 5
    GA = 3
    UN = 15
    n_main = n_chunks - NB
    assert n_main % UN == 0 and n_chunks > 2 * NB

    def body(h_hbm, src_hbm, dst_hbm, zf_hbm,
             agg_out, cnt_out,
             s0, s1, s2, s3, s4, d0, d1, d2, d3, d4, r0_, r1_, r2_,
             ones_v, zc_v, acc_sp, cnt_sp, ssem, dsem, gsem):
        srcs = [s0, s1, s2, s3, s4]
        dsts = [d0, d1, d2, d3, d4]
        rows = [r0_, r1_, r2_]
        c = lax.axis_index("c")
        s = lax.axis_index("s")
        r0 = s * rpt
        pltpu.sync_copy(zf_hbm, acc_sp.at[pl.ds(r0, rpt)])

        def zfill(i, carry):
            zc_v[pl.ds(i * 16, 16)] = jnp.zeros((16,), jnp.float32)
            return carry

        lax.fori_loop(0, rpt // 16, zfill, 0)
        for j in range(K // 16):
            ones_v[pl.ds(j * 16, 16)] = jnp.ones((16,), jnp.float32)
        pltpu.sync_copy(zc_v, cnt_sp.at[pl.ds(r0, rpt)])
        plsc.subcore_barrier()

        base = c * (E // NC) + s * e_per_tile

        def issue_loads(j, u):
            off = pl.multiple_of(base + j * K, 8)
            pltpu.async_copy(src_hbm.at[pl.ds(off, K)], srcs[u], ssem.at[u])
            pltpu.async_copy(dst_hbm.at[pl.ds(off, K)], dsts[u], dsem.at[u])

        def wait_src(u):
            pltpu.make_async_copy(src_hbm.at[pl.ds(0, K)], srcs[u],
                                  ssem.at[u]).wait()

        def wait_dst(u):
            pltpu.make_async_copy(dst_hbm.at[pl.ds(0, K)], dsts[u],
                                  dsem.at[u]).wait()

        def issue_gather(us, ur):
            pltpu.async_copy(h_hbm.at[srcs[us]], rows[ur], gsem.at[ur])

        def wait_gather(ur):
            pltpu.make_async_copy(h_hbm.at[pl.ds(0, K)], rows[ur],
                                  gsem.at[ur]).wait()

        def step(j, u5, u3):
            wait_gather(u3)
            wait_dst(u5)
            pltpu.sync_copy(rows[u3], acc_sp.at[dsts[u5]], add=True)
            pltpu.sync_copy(ones_v, cnt_sp.at[dsts[u5]], add=True)

        for u in range(NB):
            issue_loads(u, u)
        for u in range(GA):
            wait_src(u)
            issue_gather(u, u)

        def group(g, carry):
            for k in range(UN):
                j = g * UN + k
                u5, u3 = k % NB, k % GA
                step(j, u5, u3)
                issue_loads(j + NB, u5)
                wait_src((k + GA) % NB)
                issue_gather((k + GA) % NB, u3)
            return carry

        lax.fori_loop(0, n_main // UN, group, 0)

        for u in range(NB):
            j = n_main + u
            step(j, u % NB, u % GA)
            if j + GA < n_chunks:
                wait_src((u + GA) % NB)
                issue_gather((u + GA) % NB, u % GA)

        plsc.subcore_barrier()

        q0 = c * NP + r0
        pltpu.sync_copy(acc_sp.at[pl.ds(r0, rpt)], agg_out.at[pl.ds(q0, rpt)])
        pltpu.sync_copy(cnt_sp.at[pl.ds(r0, rpt)], zc_v)
        w = c * NS + s
        pltpu.sync_copy(zc_v, cnt_out.at[w])

    return pl.kernel(
        body,
        out_type=[
            jax.ShapeDtypeStruct((NC * NP, D), jnp.float32),
            jax.ShapeDtypeStruct((NC * NS, rpt), jnp.float32),
        ],
        mesh=mesh,
        scratch_types=(
            [pltpu.VMEM((K,), jnp.int32) for _ in range(2 * NB)]
            + [pltpu.VMEM((K, D), jnp.float32) for _ in range(GA)]
            + [
                pltpu.VMEM((K,), jnp.float32),
                pltpu.VMEM((rpt,), jnp.float32),
                pltpu.VMEM_SHARED((NP, D), jnp.float32),
                pltpu.VMEM_SHARED((NP,), jnp.float32),
                pltpu.SemaphoreType.DMA((NB,)),
                pltpu.SemaphoreType.DMA((NB,)),
                pltpu.SemaphoreType.DMA((GA,)),
            ]
        ),
    ), NP


def _post_body(aggp_ref, cntp_ref, x_ref, wlT_ref, bl_ref, wrT_ref, br_ref,
               o_ref):
    agg = aggp_ref[0] + aggp_ref[1]
    cnt = jnp.sum(cntp_ref[...], axis=0)
    mean = agg / jnp.maximum(cnt, 1.0)
    x = x_ref[...]
    out = (jnp.dot(mean, wlT_ref[...], preferred_element_type=jnp.float32)
           + bl_ref[...]
           + jnp.dot(x, wrT_ref[...], preferred_element_type=jnp.float32)
           + br_ref[...])
    out = jnp.where(out > 0, out, jnp.exp(jnp.minimum(out, 0.0)) - 1.0)
    norm = jnp.maximum(jnp.sqrt(jnp.sum(out * out, axis=1, keepdims=True)),
                       1e-12)
    o_ref[...] = x + out / norm


@functools.lru_cache(maxsize=None)
def _make_post(N, D, RB):
    return pl.pallas_call(
        _post_body,
        grid=(N // RB,),
        in_specs=[
            pl.BlockSpec((NC, RB, D), lambda i: (0, i, 0)),
            pl.BlockSpec((NC, RB, 1), lambda i: (0, i, 0)),
            pl.BlockSpec((RB, D), lambda i: (i, 0)),
            pl.BlockSpec((D, D), lambda i: (0, 0)),
            pl.BlockSpec((1, D), lambda i: (0, 0)),
            pl.BlockSpec((D, D), lambda i: (0, 0)),
            pl.BlockSpec((1, D), lambda i: (0, 0)),
        ],
        out_specs=pl.BlockSpec((RB, D), lambda i: (i, 0)),
        out_shape=jax.ShapeDtypeStruct((N, D), jnp.float32),
    )


def kernel(x, edge_index, W_proj, b_proj, W_l, b_l, W_r, b_r):
    N, D = x.shape
    E = edge_index.shape[1]
    src = edge_index[0].astype(jnp.int32)
    dst = edge_index[1].astype(jnp.int32)

    h = _make_proj(N, D, 1000)(x, W_proj.T, b_proj.reshape(1, D))

    sc_call, NP = _make_sc_aggregate(N, D, E)
    zf = jnp.zeros((NP // NS, D), jnp.float32)
    aggp, cntp = sc_call(h, src, dst, zf)

    out = _make_post(N, D, 1000)(
        aggp.reshape(NC, NP, D), cntp.reshape(NC, NP, 1),
        x, W_l.T, b_l.reshape(1, D), W_r.T, b_r.reshape(1, D))
    return out


# speedup vs baseline: 13.9560x; 1.0047x over previous
import functools

import jax
import jax.numpy as jnp
from jax import lax
from jax.experimental import pallas as pl
from jax.experimental.pallas import tpu as pltpu
from jax.experimental.pallas import tpu_sc as plsc

NC = 2
NS = 16
K = 80
CW = 16


def _proj_body(x_ref, wT_ref, b_ref, o_ref):
    h = jnp.dot(x_ref[...], wT_ref[...], preferred_element_type=jnp.float32)
    o_ref[...] = jnp.maximum(h + b_ref[...], 0.0)


@functools.lru_cache(maxsize=None)
def _make_proj(N, D, RB):
    return pl.pallas_call(
        _proj_body,
        grid=(N // RB,),
        in_specs=[
            pl.BlockSpec((RB, D), lambda i: (i, 0)),
            pl.BlockSpec((D, D), lambda i: (0, 0)),
            pl.BlockSpec((1, D), lambda i: (0, 0)),
        ],
        out_specs=pl.BlockSpec((RB, D), lambda i: (i, 0)),
        out_shape=jax.ShapeDtypeStruct((N, D), jnp.float32),
    )


@functools.lru_cache(maxsize=None)
def _make_sc_aggregate(N, D, E):
    NP = -(-N // (NS * 8)) * (NS * 8)
    rpt = NP // NS
    e_per_tile = E // (NC * NS)
    n_chunks = e_per_tile // K
    assert n_chunks * K == e_per_tile

    mesh = plsc.VectorSubcoreMesh(core_axis_name="c", subcore_axis_name="s")

    NB = 5
    RR = 4
    GA = 3
    LL = 4
    UN = 20
    n_main = n_chunks - NB
    assert n_main % UN == 0 and n_chunks > 2 * NB

    def body(h_hbm, src_hbm, dst_hbm, zf_hbm,
             agg_out, cnt_out,
             s0, s1, s2, s3, s4, d0, d1, d2, d3, d4, r0_, r1_, r2_, r3_,
             ones_v, zc_v, acc_sp, cnt_sp, ssem, dsem, gsem, wsem, csem):
        srcs = [s0, s1, s2, s3, s4]
        dsts = [d0, d1, d2, d3, d4]
        rows = [r0_, r1_, r2_, r3_]
        c = lax.axis_index("c")
        s = lax.axis_index("s")
        r0 = s * rpt
        pltpu.sync_copy(zf_hbm, acc_sp.at[pl.ds(r0, rpt)])

        def zfill(i, carry):
            zc_v[pl.ds(i * 16, 16)] = jnp.zeros((16,), jnp.float32)
            return carry

        lax.fori_loop(0, rpt // 16, zfill, 0)
        for j in range(K // 16):
            ones_v[pl.ds(j * 16, 16)] = jnp.ones((16,), jnp.float32)
        pltpu.sync_copy(zc_v, cnt_sp.at[pl.ds(r0, rpt)])
        plsc.subcore_barrier()

        base = c * (E // NC) + s * e_per_tile

        def issue_loads(j, u):
            off = pl.multiple_of(base + j * K, 8)
            pltpu.async_copy(src_hbm.at[pl.ds(off, K)], srcs[u], ssem.at[u])
            pltpu.async_copy(dst_hbm.at[pl.ds(off, K)], dsts[u], dsem.at[u])

        def wait_src(u):
            pltpu.make_async_copy(src_hbm.at[pl.ds(0, K)], srcs[u],
                                  ssem.at[u]).wait()

        def wait_dst(u):
            pltpu.make_async_copy(dst_hbm.at[pl.ds(0, K)], dsts[u],
                                  dsem.at[u]).wait()

        def issue_gather(us, ur):
            pltpu.async_copy(h_hbm.at[srcs[us]], rows[ur], gsem.at[ur])

        def wait_gather(us, ur):
            pltpu.make_async_copy(h_hbm.at[srcs[us]], rows[ur],
                                  gsem.at[ur]).wait()

        def wait_rowscatter(u4, u5):
            pltpu.make_async_copy(rows[u4], acc_sp.at[dsts[u5]],
                                  wsem.at[u4]).wait()

        def wait_cntscatter(u5):
            pltpu.make_async_copy(ones_v, cnt_sp.at[dsts[u5]],
                                  csem.at[u5]).wait()

        def step(j, u5, u4, first=False, do_load=True, do_gather=True):
            if not first:
                wait_rowscatter((u4 - 1) % RR, (u5 - 1) % NB)
                wait_cntscatter((u5 - 1) % NB)
            wait_gather(u5, u4)
            wait_dst(u5)
            pltpu.async_copy(rows[u4], acc_sp.at[dsts[u5]], wsem.at[u4],
                             add=True)
            pltpu.async_copy(ones_v, cnt_sp.at[dsts[u5]], csem.at[u5],
                             add=True)
            if do_load:
                issue_loads(j + LL, (u5 + LL) % NB)
            if do_gather:
                wait_src((u5 + GA) % NB)
                issue_gather((u5 + GA) % NB, (u4 + GA) % RR)

        for m in range(LL):
            issue_loads(m, m)
        for m in range(GA):
            wait_src(m)
            issue_gather(m, m)
        step(0, 0, 0, first=True)

        def group(g, carry):
            for k in range(UN):
                j = 1 + g * UN + k
                step(j, (1 + k) % NB, (1 + k) % RR)
            return carry

        lax.fori_loop(0, n_main // UN, group, 0)

        for m in range(NB - 1):
            j = n_main + 1 + m
            step(j, j % NB, j % RR, do_load=(j + LL < n_chunks),
                 do_gather=(j + GA < n_chunks))

        wait_rowscatter((n_chunks - 1) % RR, (n_chunks - 1) % NB)
        wait_cntscatter((n_chunks - 1) % NB)

        plsc.subcore_barrier()

        q0 = c * NP + r0
        pltpu.sync_copy(acc_sp.at[pl.ds(r0, rpt)], agg_out.at[pl.ds(q0, rpt)])
        pltpu.sync_copy(cnt_sp.at[pl.ds(r0, rpt)], zc_v)
        w = c * NS + s
        pltpu.sync_copy(zc_v, cnt_out.at[w])

    return pl.kernel(
        body,
        out_type=[
            jax.ShapeDtypeStruct((NC * NP, D), jnp.float32),
            jax.ShapeDtypeStruct((NC * NS, rpt), jnp.float32),
        ],
        mesh=mesh,
        scratch_types=(
            [pltpu.VMEM((K,), jnp.int32) for _ in range(2 * NB)]
            + [pltpu.VMEM((K, D), jnp.float32) for _ in range(RR)]
            + [
                pltpu.VMEM((K,), jnp.float32),
                pltpu.VMEM((rpt,), jnp.float32),
                pltpu.VMEM_SHARED((NP, D), jnp.float32),
                pltpu.VMEM_SHARED((NP,), jnp.float32),
                pltpu.SemaphoreType.DMA((NB,)),
                pltpu.SemaphoreType.DMA((NB,)),
                pltpu.SemaphoreType.DMA((RR,)),
                pltpu.SemaphoreType.DMA((RR,)),
                pltpu.SemaphoreType.DMA((NB,)),
            ]
        ),
    ), NP


def _post_body(aggp_ref, cntp_ref, x_ref, wlT_ref, bl_ref, wrT_ref, br_ref,
               o_ref):
    agg = aggp_ref[0] + aggp_ref[1]
    cnt = jnp.sum(cntp_ref[...], axis=0)
    mean = agg / jnp.maximum(cnt, 1.0)
    x = x_ref[...]
    out = (jnp.dot(mean, wlT_ref[...], preferred_element_type=jnp.float32)
           + bl_ref[...]
           + jnp.dot(x, wrT_ref[...], preferred_element_type=jnp.float32)
           + br_ref[...])
    out = jnp.where(out > 0, out, jnp.exp(jnp.minimum(out, 0.0)) - 1.0)
    norm = jnp.maximum(jnp.sqrt(jnp.sum(out * out, axis=1, keepdims=True)),
                       1e-12)
    o_ref[...] = x + out / norm


@functools.lru_cache(maxsize=None)
def _make_post(N, D, RB):
    return pl.pallas_call(
        _post_body,
        grid=(N // RB,),
        in_specs=[
            pl.BlockSpec((NC, RB, D), lambda i: (0, i, 0)),
            pl.BlockSpec((NC, RB, 1), lambda i: (0, i, 0)),
            pl.BlockSpec((RB, D), lambda i: (i, 0)),
            pl.BlockSpec((D, D), lambda i: (0, 0)),
            pl.BlockSpec((1, D), lambda i: (0, 0)),
            pl.BlockSpec((D, D), lambda i: (0, 0)),
            pl.BlockSpec((1, D), lambda i: (0, 0)),
        ],
        out_specs=pl.BlockSpec((RB, D), lambda i: (i, 0)),
        out_shape=jax.ShapeDtypeStruct((N, D), jnp.float32),
    )


def kernel(x, edge_index, W_proj, b_proj, W_l, b_l, W_r, b_r):
    N, D = x.shape
    E = edge_index.shape[1]
    src = edge_index[0].astype(jnp.int32)
    dst = edge_index[1].astype(jnp.int32)

    h = _make_proj(N, D, 1000)(x, W_proj.T, b_proj.reshape(1, D))

    sc_call, NP = _make_sc_aggregate(N, D, E)
    zf = jnp.zeros((NP // NS, D), jnp.float32)
    aggp, cntp = sc_call(h, src, dst, zf)

    out = _make_post(N, D, 1000)(
        aggp.reshape(NC, NP, D), cntp.reshape(NC, NP, 1),
        x, W_l.T, b_l.reshape(1, D), W_r.T, b_r.reshape(1, D))
    return out


# speedup vs baseline: 14.1738x; 1.0156x over previous
import functools

import jax
import jax.numpy as jnp
from jax import lax
from jax.experimental import pallas as pl
from jax.experimental.pallas import tpu as pltpu
from jax.experimental.pallas import tpu_sc as plsc

NC = 2
NS = 16
K = 80
CW = 16


def _dot_nt(a, w):
    return lax.dot_general(a, w, (((1,), (1,)), ((), ())),
                           preferred_element_type=jnp.float32)


def _proj_body(x_ref, w_ref, b_ref, o_ref):
    o_ref[...] = jnp.maximum(_dot_nt(x_ref[...], w_ref[...]) + b_ref[...],
                             0.0)


@functools.lru_cache(maxsize=None)
def _make_proj(N, D, RB):
    return pl.pallas_call(
        _proj_body,
        grid=(N // RB,),
        in_specs=[
            pl.BlockSpec((RB, D), lambda i: (i, 0)),
            pl.BlockSpec((D, D), lambda i: (0, 0)),
            pl.BlockSpec((1, D), lambda i: (0, 0)),
        ],
        out_specs=pl.BlockSpec((RB, D), lambda i: (i, 0)),
        out_shape=jax.ShapeDtypeStruct((N, D), jnp.float32),
    )


@functools.lru_cache(maxsize=None)
def _make_sc_aggregate(N, D, E):
    NP = -(-N // (NS * 8)) * (NS * 8)
    rpt = NP // NS
    e_per_tile = E // (NC * NS)
    n_chunks = e_per_tile // K
    assert n_chunks * K == e_per_tile

    mesh = plsc.VectorSubcoreMesh(core_axis_name="c", subcore_axis_name="s")

    NB = 5
    RR = 4
    GA = 3
    LL = 4
    UN = 20
    n_main = n_chunks - NB
    assert n_main % UN == 0 and n_chunks > 2 * NB

    def body(h_hbm, src_hbm, dst_hbm, zf_hbm,
             agg_out, cnt_out,
             s0, s1, s2, s3, s4, d0, d1, d2, d3, d4, r0_, r1_, r2_, r3_,
             ones_v, zc_v, acc_sp, cnt_sp, ssem, dsem, gsem, wsem, csem):
        srcs = [s0, s1, s2, s3, s4]
        dsts = [d0, d1, d2, d3, d4]
        rows = [r0_, r1_, r2_, r3_]
        c = lax.axis_index("c")
        s = lax.axis_index("s")
        r0 = s * rpt
        pltpu.sync_copy(zf_hbm, acc_sp.at[pl.ds(r0, rpt)])

        def zfill(i, carry):
            zc_v[pl.ds(i * 16, 16)] = jnp.zeros((16,), jnp.float32)
            return carry

        lax.fori_loop(0, rpt // 16, zfill, 0)
        for j in range(K // 16):
            ones_v[pl.ds(j * 16, 16)] = jnp.ones((16,), jnp.float32)
        pltpu.sync_copy(zc_v, cnt_sp.at[pl.ds(r0, rpt)])
        plsc.subcore_barrier()

        base = c * (E // NC) + s * e_per_tile

        def issue_loads(j, u):
            off = pl.multiple_of(base + j * K, 8)
            pltpu.async_copy(src_hbm.at[pl.ds(off, K)], srcs[u], ssem.at[u])
            pltpu.async_copy(dst_hbm.at[pl.ds(off, K)], dsts[u], dsem.at[u])

        def wait_src(u):
            pltpu.make_async_copy(src_hbm.at[pl.ds(0, K)], srcs[u],
                                  ssem.at[u]).wait()

        def wait_dst(u):
            pltpu.make_async_copy(dst_hbm.at[pl.ds(0, K)], dsts[u],
                                  dsem.at[u]).wait()

        def issue_gather(us, ur):
            pltpu.async_copy(h_hbm.at[srcs[us]], rows[ur], gsem.at[ur])

        def wait_gather(us, ur):
            pltpu.make_async_copy(h_hbm.at[srcs[us]], rows[ur],
                                  gsem.at[ur]).wait()

        def wait_rowscatter(u4, u5):
            pltpu.make_async_copy(rows[u4], acc_sp.at[dsts[u5]],
                                  wsem.at[u4]).wait()

        def wait_cntscatter(u5):
            pltpu.make_async_copy(ones_v, cnt_sp.at[dsts[u5]],
                                  csem.at[u5]).wait()

        def step(j, u5, u4, first=False, do_load=True, do_gather=True):
            if not first:
                wait_rowscatter((u4 - 1) % RR, (u5 - 1) % NB)
                wait_cntscatter((u5 - 1) % NB)
            wait_gather(u5, u4)
            wait_dst(u5)
            pltpu.async_copy(rows[u4], acc_sp.at[dsts[u5]], wsem.at[u4],
                             add=True)
            pltpu.async_copy(ones_v, cnt_sp.at[dsts[u5]], csem.at[u5],
                             add=True)
            if do_load:
                issue_loads(j + LL, (u5 + LL) % NB)
            if do_gather:
                wait_src((u5 + GA) % NB)
                issue_gather((u5 + GA) % NB, (u4 + GA) % RR)

        for m in range(LL):
            issue_loads(m, m)
        for m in range(GA):
            wait_src(m)
            issue_gather(m, m)
        step(0, 0, 0, first=True)

        def group(g, carry):
            for k in range(UN):
                j = 1 + g * UN + k
                step(j, (1 + k) % NB, (1 + k) % RR)
            return carry

        lax.fori_loop(0, n_main // UN, group, 0)

        for m in range(NB - 1):
            j = n_main + 1 + m
            step(j, j % NB, j % RR, do_load=(j + LL < n_chunks),
                 do_gather=(j + GA < n_chunks))

        wait_rowscatter((n_chunks - 1) % RR, (n_chunks - 1) % NB)
        wait_cntscatter((n_chunks - 1) % NB)

        plsc.subcore_barrier()

        q0 = c * NP + r0
        pltpu.sync_copy(acc_sp.at[pl.ds(r0, rpt)], agg_out.at[pl.ds(q0, rpt)])
        pltpu.sync_copy(cnt_sp.at[pl.ds(r0, rpt)], zc_v)
        w = c * NS + s
        pltpu.sync_copy(zc_v, cnt_out.at[w])

    return pl.kernel(
        body,
        out_type=[
            jax.ShapeDtypeStruct((NC * NP, D), jnp.float32),
            jax.ShapeDtypeStruct((NC * NS, rpt), jnp.float32),
        ],
        mesh=mesh,
        scratch_types=(
            [pltpu.VMEM((K,), jnp.int32) for _ in range(2 * NB)]
            + [pltpu.VMEM((K, D), jnp.float32) for _ in range(RR)]
            + [
                pltpu.VMEM((K,), jnp.float32),
                pltpu.VMEM((rpt,), jnp.float32),
                pltpu.VMEM_SHARED((NP, D), jnp.float32),
                pltpu.VMEM_SHARED((NP,), jnp.float32),
                pltpu.SemaphoreType.DMA((NB,)),
                pltpu.SemaphoreType.DMA((NB,)),
                pltpu.SemaphoreType.DMA((RR,)),
                pltpu.SemaphoreType.DMA((RR,)),
                pltpu.SemaphoreType.DMA((NB,)),
            ]
        ),
    ), NP


def _post_body(aggp_ref, cntp_ref, x_ref, wl_ref, bl_ref, wr_ref, br_ref,
               o_ref):
    agg = aggp_ref[0] + aggp_ref[1]
    cnt = jnp.sum(cntp_ref[...], axis=0)
    mean = agg / jnp.maximum(cnt, 1.0)
    x = x_ref[...]
    out = (_dot_nt(mean, wl_ref[...]) + bl_ref[...]
           + _dot_nt(x, wr_ref[...]) + br_ref[...])
    out = jnp.where(out > 0, out, jnp.exp(jnp.minimum(out, 0.0)) - 1.0)
    norm = jnp.maximum(jnp.sqrt(jnp.sum(out * out, axis=1, keepdims=True)),
                       1e-12)
    o_ref[...] = x + out / norm


@functools.lru_cache(maxsize=None)
def _make_post(N, D, RB):
    return pl.pallas_call(
        _post_body,
        grid=(N // RB,),
        in_specs=[
            pl.BlockSpec((NC, RB, D), lambda i: (0, i, 0)),
            pl.BlockSpec((NC, RB, 1), lambda i: (0, i, 0)),
            pl.BlockSpec((RB, D), lambda i: (i, 0)),
            pl.BlockSpec((D, D), lambda i: (0, 0)),
            pl.BlockSpec((1, D), lambda i: (0, 0)),
            pl.BlockSpec((D, D), lambda i: (0, 0)),
            pl.BlockSpec((1, D), lambda i: (0, 0)),
        ],
        out_specs=pl.BlockSpec((RB, D), lambda i: (i, 0)),
        out_shape=jax.ShapeDtypeStruct((N, D), jnp.float32),
    )


def kernel(x, edge_index, W_proj, b_proj, W_l, b_l, W_r, b_r):
    N, D = x.shape
    E = edge_index.shape[1]
    src = edge_index[0].astype(jnp.int32)
    dst = edge_index[1].astype(jnp.int32)

    h = _make_proj(N, D, 1000)(x, W_proj, b_proj.reshape(1, D))

    sc_call, NP = _make_sc_aggregate(N, D, E)
    zf = jnp.zeros((NP // NS, D), jnp.float32)
    aggp, cntp = sc_call(h, src, dst, zf)

    out = _make_post(N, D, 1000)(
        aggp.reshape(NC, NP, D), cntp.reshape(NC, NP, 1),
        x, W_l, b_l.reshape(1, D), W_r, b_r.reshape(1, D))
    return out
